# baseline (speedup 1.0000x reference)
import functools

import jax
import jax.numpy as jnp
from jax import lax
from jax.experimental import pallas as pl
from jax.experimental.pallas import tpu as pltpu
from jax.experimental.pallas import tpu_sc as plsc

D = 32
K = 10
LANES = 16
NC, NS = 2, 16
NW = NC * NS
NOUT = K + 1
BLKW = 128
TBLK = 244
NBIN = TBLK + 1
ICAP = 48
UCAP = 16
RING = 8
NTAB = 1000000
NSLOT = 49152
DUMMY = NSLOT
CHUNK = 128


def _iota():
    return lax.iota(jnp.int32, LANES)


def _scan_range(idx_v, bins_v, cnt_v, cap, lo_vreg, hi_vreg, wid):
    iota = _iota()
    wspl = jnp.full((LANES,), wid, jnp.int32)
    ones = jnp.ones((LANES,), jnp.int32)

    def body(i, carry):
        v = idx_v[pl.ds(i * LANES, LANES)]
        blk = v >> 7
        small = blk < NW * TBLK
        owner = jnp.where(small, blk // TBLK, blk - NW * TBLK)
        m = owner == wspl
        loc = jnp.where(small, blk - wid * TBLK, jnp.full((LANES,), TBLK,
                                                          jnp.int32))
        loc = jnp.where(m, loc, 0)
        slotv = iota + i * LANES
        pk = (slotv << 7) | (v & 127)
        nhit = plsc.all_reduce_population_count(m)[0]

        @pl.when(nhit == 1)
        def _():
            cnt = plsc.load_gather(cnt_v, [loc], mask=m)
            tgt = loc * cap + jnp.minimum(cnt, cap - 1)
            plsc.store_scatter(bins_v, [tgt], pk, mask=m)
            plsc.addupdate_scatter(cnt_v, [loc], ones, mask=m)

        @pl.when(nhit > 1)
        def _():
            key = jnp.where(m, loc, jnp.full((LANES,), 1023, jnp.int32))
            sk, sv = plsc.sort_key_val(key, pk)
            sm = sk < 1023
            ks = jnp.where(sm, sk, 0)
            prev = sk.at[jnp.maximum(iota - 1, 0)].get(
                mode="promise_in_bounds")
            nxt = sk.at[jnp.minimum(iota + 1, LANES - 1)].get(
                mode="promise_in_bounds")
            bnd = (sk != prev) | (iota == 0)
            segstart = plsc.cummax(jnp.where(bnd, iota, 0))
            rank = iota - segstart
            islast = ((iota == LANES - 1) | (sk != nxt)) & sm
            cnt = plsc.load_gather(cnt_v, [ks], mask=sm)
            tgt = ks * cap + jnp.minimum(cnt + rank, cap - 1)
            plsc.store_scatter(bins_v, [tgt], sv, mask=sm)
            plsc.addupdate_scatter(cnt_v, [ks], rank + 1, mask=islast)

        return carry

    lax.fori_loop(lo_vreg, hi_vreg, body, 0)


def _sc_extract(users_hbm, pos_hbm, neg_hbm, utab_hbm, itab_hbm, scratch_hbm,
                idx_v, ibin_v, ubin_v, icnt_v, ucnt_v,
                ulist_v, rbufs, rows_b, slots_b,
                sem_r, sem_sc, *, batch):
    wid = lax.axis_index("s") * NC + lax.axis_index("c")
    iota = _iota()
    dummy = jnp.full((LANES,), DUMMY, jnp.int32)
    nbi = TBLK + (wid < 5).astype(jnp.int32)

    def fire(tab_hbm, g_blk, r):
        col = pl.multiple_of(g_blk * BLKW, BLKW)
        pltpu.async_copy(tab_hbm.at[:, pl.ds(col, BLKW)], rbufs.at[r], sem_r)

    def gid(j):
        return jnp.where(j < TBLK, wid * TBLK + j, NW * TBLK + wid)

    for jj in range(RING):
        fire(itab_hbm, gid(jj), jj)

    pltpu.sync_copy(users_hbm.at[pl.ds(0, batch)], idx_v.at[pl.ds(0, batch)])
    pltpu.sync_copy(pos_hbm.at[pl.ds(0, batch)],
                    idx_v.at[pl.ds(batch, batch)])
    pltpu.sync_copy(neg_hbm.at[pl.ds(0, batch * K)],
                    idx_v.at[pl.ds(2 * batch, batch * K)])

    zeros16 = jnp.zeros((LANES,), jnp.int32)
    for z in range(256 // LANES):
        icnt_v[pl.ds(z * LANES, LANES)] = zeros16
        ucnt_v[pl.ds(z * LANES, LANES)] = zeros16
    for p in range(2):
        for z in range(CHUNK // LANES):
            slots_b[p, pl.ds(z * LANES, LANES)] = dummy

    nuv = batch // LANES
    ntv = NSLOT // LANES
    _scan_range(idx_v, ubin_v, ucnt_v, UCAP, 0, nuv, wid)
    _scan_range(idx_v, ibin_v, icnt_v, ICAP, nuv, ntv, wid)

    def ub(z, ul):
        c = ucnt_v[pl.ds(z * LANES, LANES)]
        m = c > 0
        blkid = iota + z * LANES
        cum = plsc.cumsum(m.astype(jnp.int32))
        pos = jnp.where(m, ul + cum - 1, 0)
        plsc.store_scatter(ulist_v, [pos], blkid, mask=m)
        return ul + plsc.all_reduce_population_count(m)[0]

    ulen = lax.fori_loop(0, 256 // LANES, ub, jnp.int32(0))

    def vscal(ref, i):
        return plsc.load_gather(ref, [jnp.full((LANES,), i, jnp.int32)])[0]

    def flush(fill, nflush):
        @pl.when(nflush >= 1)
        def _():
            pltpu.make_async_copy(rows_b.at[0],
                                  scratch_hbm.at[pl.ds(0, CHUNK)],
                                  sem_sc).wait()
        p = nflush % 2

        @pl.when(p == 0)
        def _():
            pltpu.async_copy(rows_b.at[0], scratch_hbm.at[slots_b.at[0]],
                             sem_sc)

        @pl.when(p == 1)
        def _():
            pltpu.async_copy(rows_b.at[1], scratch_hbm.at[slots_b.at[1]],
                             sem_sc)
        q = 1 - p
        qspl = jnp.full((LANES,), q, jnp.int32)
        for z in range(CHUNK // LANES):
            plsc.store_scatter(slots_b, [qspl, iota + z * LANES], dummy)
        return jnp.int32(0), nflush + 1

    def extract(bins_v, cap, rv, L, n, fill, nflush):
        def gbody(g, carry):
            fill, nflush = carry
            fill, nflush = lax.cond(fill > CHUNK - LANES, flush,
                                    lambda f, nf: (f, nf), fill, nflush)
            pk = bins_v[pl.ds(L * cap + g * LANES, LANES)]
            mrem = iota < (n - g * LANES)
            col = pk & 127
            slot = (pk >> 7) & 0xFFFF
            qspl = jnp.full((LANES,), nflush % 2, jnp.int32)
            fillpos = iota + fill

            def dstep(d, c):
                dspl = jnp.full((LANES,), d, jnp.int32)
                rspl = jnp.full((LANES,), rv, jnp.int32)
                val = plsc.load_gather(rbufs, [rspl, dspl, col], mask=mrem)
                plsc.store_scatter(rows_b, [qspl, fillpos, dspl], val,
                                   mask=mrem)
                return c

            lax.fori_loop(0, D, dstep, 0)
            plsc.store_scatter(slots_b, [qspl, fillpos], slot, mask=mrem)
            return fill + jnp.minimum(LANES, n - g * LANES), nflush

        return lax.fori_loop(0, (n + LANES - 1) // LANES, gbody,
                             (fill, nflush))

    def ibody(j, carry):
        fill, nflush = carry
        pltpu.make_async_copy(itab_hbm.at[:, pl.ds(0, BLKW)], rbufs.at[0],
                              sem_r).wait()
        n = jnp.minimum(vscal(icnt_v, j), ICAP)
        fill, nflush = extract(ibin_v, ICAP, j % RING, j, n, fill, nflush)

        @pl.when(j + RING < nbi)
        def _():
            fire(itab_hbm, gid(j + RING), (j + RING) % RING)

        return fill, nflush

    carry = lax.fori_loop(0, nbi, ibody, (jnp.int32(0), jnp.int32(0)))

    for jj in range(RING):
        @pl.when(jj < ulen)
        def _(jj=jj):
            fire(utab_hbm, gid(vscal(ulist_v, jj)), jj % RING)

    def ubody(j, carry):
        fill, nflush = carry
        pltpu.make_async_copy(utab_hbm.at[:, pl.ds(0, BLKW)], rbufs.at[0],
                              sem_r).wait()
        L = vscal(ulist_v, j)
        n = jnp.minimum(vscal(ucnt_v, L), UCAP)
        fill, nflush = extract(ubin_v, UCAP, j % RING, L, n, fill, nflush)

        @pl.when(j + RING < ulen)
        def _():
            fire(utab_hbm, gid(vscal(ulist_v, j + RING)), (j + RING) % RING)

        return fill, nflush

    carry = lax.fori_loop(0, ulen, ubody, carry)

    fill, nflush = carry
    fill, nflush = lax.cond(fill > 0, flush, lambda f, nf: (f, nf),
                            fill, nflush)

    @pl.when(nflush >= 1)
    def _():
        pltpu.make_async_copy(rows_b.at[0], scratch_hbm.at[pl.ds(0, CHUNK)],
                              sem_sc).wait()


def _sc_score(scratch_hbm, out_hbm, rows_v, out_v, sem, *, bpw):
    wid = lax.axis_index("s") * NC + lax.axis_index("c")
    base = wid * bpw
    batch = bpw * NW

    cps = [pltpu.async_copy(scratch_hbm.at[pl.ds(base, bpw)],
                            rows_v.at[pl.ds(0, bpw)], sem),
           pltpu.async_copy(scratch_hbm.at[pl.ds(batch + base, bpw)],
                            rows_v.at[pl.ds(bpw, bpw)], sem),
           pltpu.async_copy(scratch_hbm.at[pl.ds(2 * batch + base * K,
                                                 bpw * K)],
                            rows_v.at[pl.ds(2 * bpw, bpw * K)], sem)]
    for cp in cps:
        cp.wait()

    for g in range(bpw // LANES):
        b_idx = _iota() + g * LANES

        def dim_step(d, accs, b_idx=b_idx):
            dspl = jnp.full((LANES,), d, jnp.int32)
            u = plsc.load_gather(rows_v, [b_idx, dspl])
            p = plsc.load_gather(rows_v, [b_idx + bpw, dspl])
            new = [accs[0] + u * p]
            for k in range(K):
                n = plsc.load_gather(
                    rows_v, [b_idx * K + (2 * bpw + k), dspl])
                new.append(accs[k + 1] + u * n)
            return tuple(new)

        zeros = tuple(jnp.zeros((LANES,), jnp.float32) for _ in range(NOUT))
        accs = lax.fori_loop(0, D, dim_step, zeros)
        for k in range(NOUT):
            plsc.store_scatter(out_v,
                               [b_idx, jnp.full((LANES,), k, jnp.int32)],
                               accs[k])

    pltpu.sync_copy(out_v, out_hbm.at[pl.ds(base, bpw)])


def kernel(users, pos_items, neg_items, user_table, item_table):
    batch = users.shape[0]
    bpw = batch // NW
    neg_flat = neg_items.reshape(-1)
    utab_t = user_table.T
    itab_t = item_table.T

    mesh = plsc.VectorSubcoreMesh(core_axis_name="c", subcore_axis_name="s")
    params = pltpu.CompilerParams(needs_layout_passes=False,
                                  use_tc_tiling_on_sc=False)

    extract_run = functools.partial(
        pl.kernel,
        mesh=mesh,
        compiler_params=params,
        out_type=jax.ShapeDtypeStruct((NSLOT + 1, D), jnp.float32),
        scratch_types=[
            pltpu.VMEM((NSLOT,), jnp.int32),
            pltpu.VMEM((NBIN * ICAP,), jnp.int32),
            pltpu.VMEM((NBIN * UCAP,), jnp.int32),
            pltpu.VMEM((256,), jnp.int32),
            pltpu.VMEM((256,), jnp.int32),
            pltpu.VMEM((256,), jnp.int32),
            pltpu.VMEM((RING, D, BLKW), jnp.float32),
            pltpu.VMEM((2, CHUNK, D), jnp.float32),
            pltpu.VMEM((2, CHUNK), jnp.int32),
            pltpu.SemaphoreType.DMA,
            pltpu.SemaphoreType.DMA,
        ],
    )(functools.partial(_sc_extract, batch=batch))
    scratch = extract_run(users, pos_items, neg_flat, utab_t, itab_t)

    score_run = functools.partial(
        pl.kernel,
        mesh=mesh,
        compiler_params=params,
        out_type=jax.ShapeDtypeStruct((batch, NOUT), jnp.float32),
        scratch_types=[
            pltpu.VMEM((NSLOT // NW, D), jnp.float32),
            pltpu.VMEM((bpw, NOUT), jnp.float32),
            pltpu.SemaphoreType.DMA,
        ],
    )(functools.partial(_sc_score, bpw=bpw))
    return score_run(scratch)


# speedup vs baseline: 6.8665x; 6.8665x over previous
import functools

import jax
import jax.numpy as jnp
from jax import lax
from jax.experimental import pallas as pl
from jax.experimental.pallas import tpu as pltpu
from jax.experimental.pallas import tpu_sc as plsc

D = 32
K = 10
LANES = 16
NC, NS = 2, 16
NW = NC * NS
NOUT = K + 1
BLKW = 128
TBLK = 244
NBIN = TBLK + 1
ICAP = 48
UCAP = 16
RING = 6
SROW = 128
NTAB = 1000000
NSLOT = 49152
DUMMY = NSLOT
CHUNK = 128


def _iota():
    return lax.iota(jnp.int32, LANES)


def _scan_range(idx_v, bins_v, cnt_v, cap, lo_vreg, hi_vreg, wid):
    iota = _iota()
    wspl = jnp.full((LANES,), wid, jnp.int32)
    ones = jnp.ones((LANES,), jnp.int32)

    def body(i, carry):
        v = idx_v[pl.ds(i * LANES, LANES)]
        blk = v >> 7
        small = blk < NW * TBLK
        owner = jnp.where(small, blk // TBLK, blk - NW * TBLK)
        m = owner == wspl
        loc = jnp.where(small, blk - wid * TBLK, jnp.full((LANES,), TBLK,
                                                          jnp.int32))
        loc = jnp.where(m, loc, 0)
        slotv = iota + i * LANES
        pk = (slotv << 7) | (v & 127)
        nhit = plsc.all_reduce_population_count(m)[0]

        @pl.when(nhit == 1)
        def _():
            cnt = plsc.load_gather(cnt_v, [loc], mask=m)
            tgt = loc * cap + jnp.minimum(cnt, cap - 1)
            plsc.store_scatter(bins_v, [tgt], pk, mask=m)
            plsc.addupdate_scatter(cnt_v, [loc], ones, mask=m)

        @pl.when(nhit > 1)
        def _():
            key = jnp.where(m, loc, jnp.full((LANES,), 1023, jnp.int32))
            sk, sv = plsc.sort_key_val(key, pk)
            sm = sk < 1023
            ks = jnp.where(sm, sk, 0)
            prev = sk.at[jnp.maximum(iota - 1, 0)].get(
                mode="promise_in_bounds")
            nxt = sk.at[jnp.minimum(iota + 1, LANES - 1)].get(
                mode="promise_in_bounds")
            bnd = (sk != prev) | (iota == 0)
            segstart = plsc.cummax(jnp.where(bnd, iota, 0))
            rank = iota - segstart
            islast = ((iota == LANES - 1) | (sk != nxt)) & sm
            cnt = plsc.load_gather(cnt_v, [ks], mask=sm)
            tgt = ks * cap + jnp.minimum(cnt + rank, cap - 1)
            plsc.store_scatter(bins_v, [tgt], sv, mask=sm)
            plsc.addupdate_scatter(cnt_v, [ks], rank + 1, mask=islast)

        return carry

    lax.fori_loop(lo_vreg, hi_vreg, body, 0)


def _sc_extract(users_hbm, pos_hbm, neg_hbm, utab_hbm, itab_hbm, scratch_hbm,
                idx_v, ibin_v, ubin_v, icnt_v, ucnt_v,
                ulist_v, rbufs, rows_b, slots_b,
                sem_r, sem_sc, *, batch):
    wid = lax.axis_index("s") * NC + lax.axis_index("c")
    iota = _iota()
    dummy = jnp.full((LANES,), DUMMY, jnp.int32)
    nbi = TBLK + (wid < 5).astype(jnp.int32)

    def fire(tab_hbm, g_blk, r):
        col = pl.multiple_of(g_blk * BLKW, BLKW)
        pltpu.async_copy(tab_hbm.at[:, pl.ds(col, BLKW)], rbufs.at[r], sem_r)

    def gid(j):
        return jnp.where(j < TBLK, wid * TBLK + j, NW * TBLK + wid)

    for jj in range(RING):
        fire(itab_hbm, gid(jj), jj)

    pltpu.sync_copy(users_hbm.at[pl.ds(0, batch)], idx_v.at[pl.ds(0, batch)])
    pltpu.sync_copy(pos_hbm.at[pl.ds(0, batch)],
                    idx_v.at[pl.ds(batch, batch)])
    pltpu.sync_copy(neg_hbm.at[pl.ds(0, batch * K)],
                    idx_v.at[pl.ds(2 * batch, batch * K)])

    zeros16 = jnp.zeros((LANES,), jnp.int32)
    for z in range(256 // LANES):
        icnt_v[pl.ds(z * LANES, LANES)] = zeros16
        ucnt_v[pl.ds(z * LANES, LANES)] = zeros16
    for p in range(2):
        for z in range(CHUNK // LANES):
            slots_b[p, pl.ds(z * LANES, LANES)] = dummy

    nuv = batch // LANES
    ntv = NSLOT // LANES
    _scan_range(idx_v, ubin_v, ucnt_v, UCAP, 0, nuv, wid)
    _scan_range(idx_v, ibin_v, icnt_v, ICAP, nuv, ntv, wid)

    def ub(z, ul):
        c = ucnt_v[pl.ds(z * LANES, LANES)]
        m = c > 0
        blkid = iota + z * LANES
        cum = plsc.cumsum(m.astype(jnp.int32))
        pos = jnp.where(m, ul + cum - 1, 0)
        plsc.store_scatter(ulist_v, [pos], blkid, mask=m)
        return ul + plsc.all_reduce_population_count(m)[0]

    ulen = lax.fori_loop(0, 256 // LANES, ub, jnp.int32(0))

    def vscal(ref, i):
        return plsc.load_gather(ref, [jnp.full((LANES,), i, jnp.int32)])[0]

    def flush(fill, nflush):
        @pl.when(nflush >= 1)
        def _():
            pltpu.make_async_copy(rows_b.at[0],
                                  scratch_hbm.at[pl.ds(0, CHUNK)],
                                  sem_sc).wait()
        p = nflush % 2

        @pl.when(p == 0)
        def _():
            pltpu.async_copy(rows_b.at[0], scratch_hbm.at[slots_b.at[0]],
                             sem_sc)

        @pl.when(p == 1)
        def _():
            pltpu.async_copy(rows_b.at[1], scratch_hbm.at[slots_b.at[1]],
                             sem_sc)
        q = 1 - p
        qspl = jnp.full((LANES,), q, jnp.int32)
        for z in range(CHUNK // LANES):
            plsc.store_scatter(slots_b, [qspl, iota + z * LANES], dummy)
        return jnp.int32(0), nflush + 1

    def extract(bins_v, cap, rv, L, n, fill, nflush):
        def gbody(g, carry):
            fill, nflush = carry
            fill, nflush = lax.cond(fill > CHUNK - LANES, flush,
                                    lambda f, nf: (f, nf), fill, nflush)
            pk = bins_v[pl.ds(L * cap + g * LANES, LANES)]
            mrem = iota < (n - g * LANES)
            col = pk & 127
            slot = (pk >> 7) & 0xFFFF
            qspl = jnp.full((LANES,), nflush % 2, jnp.int32)
            fillpos = iota + fill

            def dstep(d, c):
                dspl = jnp.full((LANES,), d, jnp.int32)
                rspl = jnp.full((LANES,), rv, jnp.int32)
                val = plsc.load_gather(rbufs, [rspl, dspl, col], mask=mrem)
                plsc.store_scatter(rows_b, [qspl, fillpos, dspl], val,
                                   mask=mrem)
                return c

            lax.fori_loop(0, D, dstep, 0)
            plsc.store_scatter(slots_b, [qspl, fillpos], slot, mask=mrem)
            return fill + jnp.minimum(LANES, n - g * LANES), nflush

        return lax.fori_loop(0, (n + LANES - 1) // LANES, gbody,
                             (fill, nflush))

    def ibody(j, carry):
        fill, nflush = carry
        pltpu.make_async_copy(itab_hbm.at[:, pl.ds(0, BLKW)], rbufs.at[0],
                              sem_r).wait()
        n = jnp.minimum(vscal(icnt_v, j), ICAP)
        fill, nflush = extract(ibin_v, ICAP, j % RING, j, n, fill, nflush)

        @pl.when(j + RING < nbi)
        def _():
            fire(itab_hbm, gid(j + RING), (j + RING) % RING)

        return fill, nflush

    carry = lax.fori_loop(0, nbi, ibody, (jnp.int32(0), jnp.int32(0)))

    for jj in range(RING):
        @pl.when(jj < ulen)
        def _(jj=jj):
            fire(utab_hbm, gid(vscal(ulist_v, jj)), jj % RING)

    def ubody(j, carry):
        fill, nflush = carry
        pltpu.make_async_copy(utab_hbm.at[:, pl.ds(0, BLKW)], rbufs.at[0],
                              sem_r).wait()
        L = vscal(ulist_v, j)
        n = jnp.minimum(vscal(ucnt_v, L), UCAP)
        fill, nflush = extract(ubin_v, UCAP, j % RING, L, n, fill, nflush)

        @pl.when(j + RING < ulen)
        def _():
            fire(utab_hbm, gid(vscal(ulist_v, j + RING)), (j + RING) % RING)

        return fill, nflush

    carry = lax.fori_loop(0, ulen, ubody, carry)

    fill, nflush = carry
    fill, nflush = lax.cond(fill > 0, flush, lambda f, nf: (f, nf),
                            fill, nflush)

    @pl.when(nflush >= 1)
    def _():
        pltpu.make_async_copy(rows_b.at[0], scratch_hbm.at[pl.ds(0, CHUNK)],
                              sem_sc).wait()


def _sc_score(scratch_hbm, out_hbm, rows_v, out_v, sem, *, bpw):
    wid = lax.axis_index("s") * NC + lax.axis_index("c")
    batch = bpw * NW
    half = bpw // 2
    hneg = half * K

    for h in range(2):
        base = wid * bpw + h * half
        cps = [pltpu.async_copy(scratch_hbm.at[pl.ds(base, half)],
                                rows_v.at[pl.ds(0, half)], sem),
               pltpu.async_copy(scratch_hbm.at[pl.ds(batch + base, half)],
                                rows_v.at[pl.ds(half, half)], sem),
               pltpu.async_copy(
                   scratch_hbm.at[pl.ds(2 * batch + base * K, hneg)],
                   rows_v.at[pl.ds(2 * half, hneg)], sem)]
        for cp in cps:
            cp.wait()

        for g in range(half // LANES):
            b_idx = _iota() + g * LANES

            def dim_step(d, accs, b_idx=b_idx):
                dspl = jnp.full((LANES,), d, jnp.int32)
                u = plsc.load_gather(rows_v, [b_idx, dspl])
                p = plsc.load_gather(rows_v, [b_idx + half, dspl])
                new = [accs[0] + u * p]
                for k in range(K):
                    n = plsc.load_gather(
                        rows_v, [b_idx * K + (2 * half + k), dspl])
                    new.append(accs[k + 1] + u * n)
                return tuple(new)

            zeros = tuple(jnp.zeros((LANES,), jnp.float32)
                          for _ in range(NOUT))
            accs = lax.fori_loop(0, D, dim_step, zeros)
            for k in range(NOUT):
                plsc.store_scatter(
                    out_v,
                    [b_idx + h * half, jnp.full((LANES,), k, jnp.int32)],
                    accs[k])

    pltpu.sync_copy(out_v, out_hbm.at[pl.ds(wid * bpw, bpw)])


def kernel(users, pos_items, neg_items, user_table, item_table):
    batch = users.shape[0]
    bpw = batch // NW
    neg_flat = neg_items.reshape(-1)
    utab_t = user_table.T
    itab_t = item_table.T

    mesh = plsc.VectorSubcoreMesh(core_axis_name="c", subcore_axis_name="s")
    params = pltpu.CompilerParams(needs_layout_passes=False,
                                  use_tc_tiling_on_sc=True)

    extract_run = functools.partial(
        pl.kernel,
        mesh=mesh,
        compiler_params=params,
        out_type=jax.ShapeDtypeStruct((NSLOT + 1, SROW), jnp.float32),
        scratch_types=[
            pltpu.VMEM((NSLOT,), jnp.int32),
            pltpu.VMEM((NBIN * ICAP,), jnp.int32),
            pltpu.VMEM((NBIN * UCAP,), jnp.int32),
            pltpu.VMEM((256,), jnp.int32),
            pltpu.VMEM((256,), jnp.int32),
            pltpu.VMEM((256,), jnp.int32),
            pltpu.VMEM((RING, D, BLKW), jnp.float32),
            pltpu.VMEM((2, CHUNK, SROW), jnp.float32),
            pltpu.VMEM((2, CHUNK), jnp.int32),
            pltpu.SemaphoreType.DMA,
            pltpu.SemaphoreType.DMA,
        ],
    )(functools.partial(_sc_extract, batch=batch))
    scratch = extract_run(users, pos_items, neg_flat, utab_t, itab_t)

    score_run = functools.partial(
        pl.kernel,
        mesh=mesh,
        compiler_params=params,
        out_type=jax.ShapeDtypeStruct((batch, SROW), jnp.float32),
        scratch_types=[
            pltpu.VMEM((NSLOT // NW // 2, SROW), jnp.float32),
            pltpu.VMEM((bpw, SROW), jnp.float32),
            pltpu.SemaphoreType.DMA,
        ],
    )(functools.partial(_sc_score, bpw=bpw))
    return score_run(scratch)[:, :NOUT]
